# baseline (speedup 1.0000x reference)
import functools

import jax
import jax.numpy as jnp
from jax import lax
from jax.experimental import pallas as pl
from jax.experimental.pallas import tpu as pltpu
from jax.experimental.pallas import tpu_sc as plsc

_TILE_R = 512
_STRIP = _TILE_R + 128


def _band_kernel(len_ref, s2b_ref):
    b = pl.program_id(0)
    i = pl.program_id(1)
    tc = len_ref[b]

    TR = s2b_ref.shape[1]
    T = s2b_ref.shape[2]
    SW = _STRIP
    row0 = i * TR
    col_lo = pl.multiple_of(jnp.maximum(row0 - 128, 0), 128)

    s2b_ref[0] = jnp.zeros((TR, T), jnp.float32)
    rs = row0 + jax.lax.broadcasted_iota(jnp.int32, (TR, SW), 0)
    cs = col_lo + jax.lax.broadcasted_iota(jnp.int32, (TR, SW), 1)
    d = rs - cs

    @pl.when(tc > 2)
    def _():
        band = jnp.where(
            d == 1,
            jnp.float32(0.8),
            jnp.where((d == 0) | (d == 2), jnp.float32(0.1), jnp.float32(0.0)),
        )
        s2b_ref[0, :, pl.ds(col_lo, SW)] = jnp.where(
            cs < tc - 2, band, jnp.float32(0.0)
        )

    @pl.when(tc <= 2)
    def _():
        s2b_ref[0, :, pl.ds(col_lo, SW)] = jnp.where(
            (d == 0) & (rs < tc), jnp.float32(1.0), jnp.float32(0.0)
        )


def _dense_kernel(len_ref, seq_ref, s2b_ref, avs_ref):
    _band_kernel(len_ref, s2b_ref)
    _blur_part(len_ref, seq_ref, avs_ref)


def _blur_part(len_ref, seq_ref, avs_ref):
    b = pl.program_id(0)
    i = pl.program_id(1)
    tc = len_ref[b]
    T = seq_ref.shape[1]
    D = seq_ref.shape[2]
    TR = avs_ref.shape[1]
    row0 = pl.multiple_of(i * TR, TR)

    x = seq_ref[0, pl.ds(row0, TR), :]

    @pl.when(tc > 2)
    def _():
        q = seq_ref[0, pl.ds(pl.multiple_of(jnp.minimum(row0 + TR, T - 8), 8), 8), :]
        ri = jax.lax.broadcasted_iota(jnp.int32, (TR, 1), 0)
        x1 = jnp.where(ri == TR - 1, q[0:1, :], jnp.roll(x, -1, axis=0))
        x2 = jnp.where(
            ri == TR - 2,
            q[0:1, :],
            jnp.where(ri == TR - 1, q[1:2, :], jnp.roll(x, -2, axis=0)),
        )
        blurred = 0.1 * x + 0.8 * x1 + 0.1 * x2
        avs_ref[0] = jnp.where(row0 + ri < tc - 2, blurred, jnp.float32(0.0))

    @pl.when(tc <= 2)
    def _():
        avs_ref[0] = x


def _make_ramp_kernel(B, T):
    info = plsc.get_sparse_core_info()
    NC, NS, L = info.num_cores, info.num_subcores, info.num_lanes
    NW = NC * NS
    WPB = NW // B
    SEG = T // WPB
    NCHUNK = SEG // L

    mesh = plsc.VectorSubcoreMesh(core_axis_name="c", subcore_axis_name="s")

    @functools.partial(
        pl.kernel,
        mesh=mesh,
        out_type=[
            jax.ShapeDtypeStruct((B, T), jnp.float32),
            jax.ShapeDtypeStruct((B, T), jnp.float32),
            jax.ShapeDtypeStruct((B,), jnp.int32),
        ],
        scratch_types=[
            pltpu.VMEM((16,), jnp.int32),
            pltpu.VMEM((16,), jnp.int32),
            pltpu.VMEM((SEG,), jnp.float32),
            pltpu.VMEM((SEG,), jnp.float32),
        ],
    )
    def ramp_kernel(len_hbm, r_hbm, ar_hbm, al_hbm, len_v, al_v, r_v, ar_v):
        wid = lax.axis_index("s") * NC + lax.axis_index("c")
        b = wid // WPB
        base = (wid % WPB) * SEG

        pltpu.sync_copy(len_hbm, len_v.at[pl.ds(0, B)])
        lane = lax.iota(jnp.int32, L)
        lv0 = len_v[...]

        for bb in range(B):

            @pl.when(b == bb)
            def _(bb=bb):
                tc = jnp.full((L,), lv0[bb], jnp.int32)
                tcf = tc.astype(jnp.float32)
                safe_tc = jnp.maximum(tcf, jnp.float32(1.0))
                safe_tc2 = jnp.maximum(tcf - 2.0, jnp.float32(1.0))
                one = jnp.full((L,), 1, jnp.int32)
                zero = jnp.full((L,), 0, jnp.int32)
                for j in range(NCHUNK):
                    t = base + j * L + lane
                    tf1 = (t + 1).astype(jnp.float32)
                    m_r = jnp.minimum(jnp.maximum(tc - t, zero), one)
                    m_ar = jnp.minimum(jnp.maximum((tc - 2) - t, zero), one)
                    g = jnp.minimum(jnp.maximum(tc - 2, zero), one)
                    gf = g.astype(jnp.float32)
                    r = tf1 / safe_tc * m_r.astype(jnp.float32)
                    ar2 = tf1 / safe_tc2 * m_ar.astype(jnp.float32)
                    ar = gf * ar2 + (1.0 - gf) * r
                    r_v[pl.ds(j * L, L)] = r
                    ar_v[pl.ds(j * L, L)] = ar

        pltpu.sync_copy(r_v, r_hbm.at[b, pl.ds(base, SEG)])
        pltpu.sync_copy(ar_v, ar_hbm.at[b, pl.ds(base, SEG)])

        @pl.when(wid == 0)
        def _():
            lv = len_v[...]
            zero16 = jnp.full((L,), 0, jnp.int32)
            one16 = jnp.full((L,), 1, jnp.int32)
            g = jnp.minimum(jnp.maximum(lv - 2, zero16), one16)
            al_v[...] = lv - 2 * g
            pltpu.sync_copy(al_v.at[pl.ds(0, B)], al_hbm)

    return ramp_kernel


def kernel(seq, len_seq):
    B, T, D = seq.shape
    TR = _TILE_R
    NR = T // TR

    s2b, avs = pl.pallas_call(
        _dense_kernel,
        grid_spec=pltpu.PrefetchScalarGridSpec(
            num_scalar_prefetch=1,
            grid=(B, NR),
            in_specs=[
                pl.BlockSpec((1, T, D), lambda b, i, L: (b, 0, 0)),
            ],
            out_specs=[
                pl.BlockSpec((1, TR, T), lambda b, i, L: (b, i, 0)),
                pl.BlockSpec((1, TR, D), lambda b, i, L: (b, i, 0)),
            ],
        ),
        out_shape=[
            jax.ShapeDtypeStruct((B, T, T), jnp.float32),
            jax.ShapeDtypeStruct((B, T, D), jnp.float32),
        ],
        compiler_params=pltpu.CompilerParams(
            dimension_semantics=("arbitrary", "arbitrary"),
        ),
    )(len_seq, seq)

    r2, ar2, al = _make_ramp_kernel(B, T)(len_seq)

    return (s2b, avs, r2, ar2, al)


# speedup vs baseline: 1.0705x; 1.0705x over previous
import functools

import jax
import jax.numpy as jnp
from jax import lax
from jax.experimental import pallas as pl
from jax.experimental.pallas import tpu as pltpu
from jax.experimental.pallas import tpu_sc as plsc

_TILE_R = 512
_STRIP = _TILE_R + 128


def _band_kernel(len_ref, s2b_ref):
    b = pl.program_id(0)
    i = pl.program_id(1)
    tc = len_ref[b]

    TR = s2b_ref.shape[1]
    T = s2b_ref.shape[2]
    SW = _STRIP
    row0 = i * TR
    col_lo = pl.multiple_of(jnp.maximum(row0 - 128, 0), 128)

    s2b_ref[0] = jnp.zeros((TR, T), jnp.float32)
    rs = row0 + jax.lax.broadcasted_iota(jnp.int32, (TR, SW), 0)
    cs = col_lo + jax.lax.broadcasted_iota(jnp.int32, (TR, SW), 1)
    d = rs - cs

    @pl.when(tc > 2)
    def _():
        band = jnp.where(
            d == 1,
            jnp.float32(0.8),
            jnp.where((d == 0) | (d == 2), jnp.float32(0.1), jnp.float32(0.0)),
        )
        s2b_ref[0, :, pl.ds(col_lo, SW)] = jnp.where(
            cs < tc - 2, band, jnp.float32(0.0)
        )

    @pl.when(tc <= 2)
    def _():
        s2b_ref[0, :, pl.ds(col_lo, SW)] = jnp.where(
            (d == 0) & (rs < tc), jnp.float32(1.0), jnp.float32(0.0)
        )


def _blur_kernel(len_ref, seq_ref, avs_ref):
    b = pl.program_id(0)
    tc = len_ref[b]

    @pl.when(tc > 2)
    def _():
        x = seq_ref[0]
        x1 = jnp.roll(x, -1, axis=0)
        x2 = jnp.roll(x, -2, axis=0)
        blurred = 0.1 * x + 0.8 * x1 + 0.1 * x2
        T = seq_ref.shape[1]
        t_col = jax.lax.broadcasted_iota(jnp.int32, (T, 1), 0)
        avs_ref[0] = jnp.where(t_col < tc - 2, blurred, jnp.float32(0.0))

    @pl.when(tc <= 2)
    def _():
        avs_ref[0] = seq_ref[0]


def _make_ramp_kernel(B, T):
    info = plsc.get_sparse_core_info()
    NC, NS, L = info.num_cores, info.num_subcores, info.num_lanes
    NW = NC * NS
    WPB = NW // B
    SEG = T // WPB
    NCHUNK = SEG // L

    mesh = plsc.VectorSubcoreMesh(core_axis_name="c", subcore_axis_name="s")

    @functools.partial(
        pl.kernel,
        mesh=mesh,
        out_type=[
            jax.ShapeDtypeStruct((B, T), jnp.float32),
            jax.ShapeDtypeStruct((B, T), jnp.float32),
            jax.ShapeDtypeStruct((B,), jnp.int32),
        ],
        scratch_types=[
            pltpu.VMEM((16,), jnp.int32),
            pltpu.VMEM((16,), jnp.int32),
            pltpu.VMEM((SEG,), jnp.float32),
            pltpu.VMEM((SEG,), jnp.float32),
        ],
    )
    def ramp_kernel(len_hbm, r_hbm, ar_hbm, al_hbm, len_v, al_v, r_v, ar_v):
        wid = lax.axis_index("s") * NC + lax.axis_index("c")
        b = wid // WPB
        base = (wid % WPB) * SEG

        pltpu.sync_copy(len_hbm, len_v.at[pl.ds(0, B)])
        lane = lax.iota(jnp.int32, L)
        lv0 = len_v[...]

        for bb in range(B):

            @pl.when(b == bb)
            def _(bb=bb):
                tc = jnp.full((L,), lv0[bb], jnp.int32)
                tcf = tc.astype(jnp.float32)
                safe_tc = jnp.maximum(tcf, jnp.float32(1.0))
                safe_tc2 = jnp.maximum(tcf - 2.0, jnp.float32(1.0))
                one = jnp.full((L,), 1, jnp.int32)
                zero = jnp.full((L,), 0, jnp.int32)
                for j in range(NCHUNK):
                    t = base + j * L + lane
                    tf1 = (t + 1).astype(jnp.float32)
                    m_r = jnp.minimum(jnp.maximum(tc - t, zero), one)
                    m_ar = jnp.minimum(jnp.maximum((tc - 2) - t, zero), one)
                    g = jnp.minimum(jnp.maximum(tc - 2, zero), one)
                    gf = g.astype(jnp.float32)
                    r = tf1 / safe_tc * m_r.astype(jnp.float32)
                    ar2 = tf1 / safe_tc2 * m_ar.astype(jnp.float32)
                    ar = gf * ar2 + (1.0 - gf) * r
                    r_v[pl.ds(j * L, L)] = r
                    ar_v[pl.ds(j * L, L)] = ar

        pltpu.sync_copy(r_v, r_hbm.at[b, pl.ds(base, SEG)])
        pltpu.sync_copy(ar_v, ar_hbm.at[b, pl.ds(base, SEG)])

        @pl.when(wid == 0)
        def _():
            lv = len_v[...]
            zero16 = jnp.full((L,), 0, jnp.int32)
            one16 = jnp.full((L,), 1, jnp.int32)
            g = jnp.minimum(jnp.maximum(lv - 2, zero16), one16)
            al_v[...] = lv - 2 * g
            pltpu.sync_copy(al_v.at[pl.ds(0, B)], al_hbm)

    return ramp_kernel


def kernel(seq, len_seq):
    B, T, D = seq.shape
    TR = _TILE_R
    NR = T // TR

    s2b = pl.pallas_call(
        _band_kernel,
        grid_spec=pltpu.PrefetchScalarGridSpec(
            num_scalar_prefetch=1,
            grid=(B, NR),
            in_specs=[],
            out_specs=[
                pl.BlockSpec((1, TR, T), lambda b, i, L: (b, i, 0)),
            ],
        ),
        out_shape=[jax.ShapeDtypeStruct((B, T, T), jnp.float32)],
        compiler_params=pltpu.CompilerParams(
            dimension_semantics=("arbitrary", "arbitrary"),
        ),
    )(len_seq)[0]

    avs = pl.pallas_call(
        _blur_kernel,
        grid_spec=pltpu.PrefetchScalarGridSpec(
            num_scalar_prefetch=1,
            grid=(B,),
            in_specs=[
                pl.BlockSpec((1, T, D), lambda b, L: (b, 0, 0)),
            ],
            out_specs=[
                pl.BlockSpec((1, T, D), lambda b, L: (b, 0, 0)),
            ],
        ),
        out_shape=[jax.ShapeDtypeStruct((B, T, D), jnp.float32)],
        compiler_params=pltpu.CompilerParams(
            dimension_semantics=("arbitrary",),
        ),
    )(len_seq, seq)[0]

    r2, ar2, al = _make_ramp_kernel(B, T)(len_seq)

    return (s2b, avs, r2, ar2, al)
